# baseline (speedup 1.0000x reference)
import functools

import jax
import jax.numpy as jnp
from jax import lax
from jax.experimental import pallas as pl
from jax.experimental.pallas import tpu as pltpu
from jax.experimental.pallas import tpu_sc as plsc


def _att_idx_body(*refs, buf, chunk, nstream):
    q_ref = refs[0]
    wks = refs[1:1 + nstream]
    us = refs[1 + nstream:1 + 2 * nstream]
    idx_ref = refs[1 + 2 * nstream]
    bz_ref, bi_ref = refs[-2], refs[-1]
    k = pl.program_id(0)
    nchunk = pl.num_programs(0)
    qv = q_ref[...]
    big = jnp.int32(2**30)

    zs = []
    for s in range(nstream):
        att = lax.dot_general(
            qv, wks[s][...],
            dimension_numbers=(((1,), (1,)), ((), ())),
            preferred_element_type=jnp.float32,
        )
        zs.append(att - jnp.log(-jnp.log(us[s][...])))
    b = zs[0].shape[0]
    mc = jnp.max(zs[0], axis=1, keepdims=True)
    for s in range(1, nstream):
        mc = jnp.maximum(mc, jnp.max(zs[s], axis=1, keepdims=True))
    ic = jnp.full((b, 1), big, jnp.int32)
    for s in range(nstream):
        ii = lax.broadcasted_iota(jnp.int32, (b, chunk), 1) \
            + (k * nstream + s) * chunk
        ics = jnp.min(jnp.where(zs[s] >= mc, ii, big), axis=1, keepdims=True)
        ic = jnp.minimum(ic, ics)

    @pl.when(k == 0)
    def _():
        bz_ref[...] = mc
        bi_ref[...] = ic

    @pl.when(k > 0)
    def _():
        better = mc > bz_ref[...]
        bz_ref[...] = jnp.where(better, mc, bz_ref[...])
        bi_ref[...] = jnp.where(better, ic, bi_ref[...])

    @pl.when(k == nchunk - 1)
    def _():
        bufd = idx_ref.shape[1]
        jj = lax.broadcasted_iota(jnp.int32, (b, bufd), 1)
        idx_ref[...] = jj * jnp.int32(buf) + bi_ref[...]


def _adain_body(x_ref, dm_ref, ds_ref, o_ref):
    xb = x_ref[...]
    hw = xb.shape[2] * xb.shape[3]
    m = jnp.sum(xb, axis=(2, 3), keepdims=True) * (1.0 / hw)
    m2 = jnp.sum(xb * xb, axis=(2, 3), keepdims=True) * (1.0 / hw)
    v = m2 - m * m
    s = ds_ref[...] * lax.rsqrt(v + 1e-5)
    o_ref[...] = xb * s + (dm_ref[...] - m * s)


def _make_sc_gather(n_elems, per_w, nc, ns):
    mesh = plsc.VectorSubcoreMesh(core_axis_name="c", subcore_axis_name="s")

    @functools.partial(
        pl.kernel,
        mesh=mesh,
        out_type=jax.ShapeDtypeStruct((n_elems,), jnp.float32),
        scratch_types=[
            pltpu.VMEM((per_w,), jnp.int32),
            pltpu.VMEM((per_w,), jnp.float32),
            pltpu.SemaphoreType.DMA,
        ],
    )
    def gather_k(wv_hbm, idx_hbm, out_hbm, idx_v, vals_v, sem):
        wid = lax.axis_index("s") * nc + lax.axis_index("c")
        base = wid * per_w
        pltpu.sync_copy(idx_hbm.at[pl.ds(base, per_w)], idx_v)
        pltpu.async_copy(wv_hbm.at[idx_v], vals_v, sem).wait()
        pltpu.sync_copy(vals_v, out_hbm.at[pl.ds(base, per_w)])

    return gather_k


def kernel(x, q, mean, std, Wk, Wv):
    b, c, h, w = x.shape
    buf, feat = Wk.shape
    bufd = Wv.shape[0]

    m_start = jnp.asarray(mean[1], jnp.int32) - c
    s_start = jnp.asarray(std[1], jnp.int32) - c
    m_start = m_start + (jnp.asarray(mean[0], jnp.int32) - m_start)
    s_start = s_start + (jnp.asarray(std[0], jnp.int32) - s_start)

    u = jax.random.uniform(jax.random.key(42), (b, buf), minval=1e-10, maxval=1.0)

    chunk = 2048
    nstream = 2
    nchunk = buf // (chunk * nstream)
    wk_specs = [
        pl.BlockSpec((chunk, feat), lambda k, s=s: (k * nstream + s, 0))
        for s in range(nstream)
    ]
    u_specs = [
        pl.BlockSpec((b, chunk), lambda k, s=s: (0, k * nstream + s))
        for s in range(nstream)
    ]
    idx_j = pl.pallas_call(
        functools.partial(_att_idx_body, buf=buf, chunk=chunk, nstream=nstream),
        grid=(nchunk,),
        in_specs=[pl.BlockSpec((b, feat), lambda k: (0, 0))]
        + wk_specs + u_specs,
        out_specs=pl.BlockSpec((b, bufd), lambda k: (0, 0)),
        out_shape=jax.ShapeDtypeStruct((b, bufd), jnp.int32),
        scratch_shapes=[
            pltpu.VMEM((b, 1), jnp.float32),
            pltpu.VMEM((b, 1), jnp.int32),
        ],
    )(q, *([Wk] * nstream), *([u] * nstream))

    n_elems = bufd * b
    info = plsc.get_sparse_core_info()
    nc, ns = info.num_cores, info.num_subcores
    per_w = n_elems // (nc * ns)
    vals = _make_sc_gather(n_elems, per_w, nc, ns)(
        Wv.reshape(-1), idx_j.reshape(-1)
    )
    ct = 48
    try:
        mo, so = int(m_start), int(s_start)
    except Exception:
        mo = so = None
    if mo is not None and mo % ct == 0 and so % ct == 0:
        dm_arr = ds_arr = vals.reshape(b, bufd, 1, 1)
        dm_spec = pl.BlockSpec((1, ct, 1, 1), lambda i, j: (i, mo // ct + j, 0, 0))
        ds_spec = pl.BlockSpec((1, ct, 1, 1), lambda i, j: (i, so // ct + j, 0, 0))
    else:
        di = vals.reshape(b, bufd)
        dm_arr = lax.dynamic_slice_in_dim(di, m_start, c, axis=1).reshape(b, c, 1, 1)
        ds_arr = lax.dynamic_slice_in_dim(di, s_start, c, axis=1).reshape(b, c, 1, 1)
        dm_spec = pl.BlockSpec((1, ct, 1, 1), lambda i, j: (i, j, 0, 0))
        ds_spec = pl.BlockSpec((1, ct, 1, 1), lambda i, j: (i, j, 0, 0))
    out = pl.pallas_call(
        _adain_body,
        grid=(b, c // ct),
        compiler_params=pltpu.CompilerParams(
            dimension_semantics=("parallel", "parallel"),
        ),
        in_specs=[
            pl.BlockSpec((1, ct, h, w), lambda i, j: (i, j, 0, 0)),
            dm_spec,
            ds_spec,
        ],
        out_specs=pl.BlockSpec((1, ct, h, w), lambda i, j: (i, j, 0, 0)),
        out_shape=jax.ShapeDtypeStruct((b, c, h, w), jnp.float32),
    )(x, dm_arr, ds_arr)
    return out


# speedup vs baseline: 1.0019x; 1.0019x over previous
import functools

import jax
import jax.numpy as jnp
from jax import lax
from jax.experimental import pallas as pl
from jax.experimental.pallas import tpu as pltpu
from jax.experimental.pallas import tpu_sc as plsc


def _att_idx_body(*refs, buf, chunk, nstream):
    q_ref = refs[0]
    wks = refs[1:1 + nstream]
    us = refs[1 + nstream:1 + 2 * nstream]
    idx_ref = refs[1 + 2 * nstream]
    bz_ref, bi_ref = refs[-2], refs[-1]
    k = pl.program_id(0)
    nchunk = pl.num_programs(0)
    qv = q_ref[...]
    big = jnp.int32(2**30)

    zs = []
    for s in range(nstream):
        att = lax.dot_general(
            qv, wks[s][...],
            dimension_numbers=(((1,), (1,)), ((), ())),
            preferred_element_type=jnp.float32,
        )
        zs.append(att - jnp.log(-jnp.log(us[s][...])))
    b = zs[0].shape[0]
    mc = jnp.max(zs[0], axis=1, keepdims=True)
    for s in range(1, nstream):
        mc = jnp.maximum(mc, jnp.max(zs[s], axis=1, keepdims=True))
    ic = jnp.full((b, 1), big, jnp.int32)
    for s in range(nstream):
        ii = lax.broadcasted_iota(jnp.int32, (b, chunk), 1) \
            + (k * nstream + s) * chunk
        ics = jnp.min(jnp.where(zs[s] >= mc, ii, big), axis=1, keepdims=True)
        ic = jnp.minimum(ic, ics)

    @pl.when(k == 0)
    def _():
        bz_ref[...] = mc
        bi_ref[...] = ic

    @pl.when(k > 0)
    def _():
        better = mc > bz_ref[...]
        bz_ref[...] = jnp.where(better, mc, bz_ref[...])
        bi_ref[...] = jnp.where(better, ic, bi_ref[...])

    @pl.when(k == nchunk - 1)
    def _():
        bufd = idx_ref.shape[1]
        jj = lax.broadcasted_iota(jnp.int32, (b, bufd), 1)
        idx_ref[...] = jj * jnp.int32(buf) + bi_ref[...]


def _adain_body(x_ref, dm_ref, ds_ref, o_ref):
    xb = x_ref[...]
    hw = xb.shape[2] * xb.shape[3]
    m = jnp.sum(xb, axis=(2, 3), keepdims=True) * (1.0 / hw)
    m2 = jnp.sum(xb * xb, axis=(2, 3), keepdims=True) * (1.0 / hw)
    v = m2 - m * m
    s = ds_ref[...] * lax.rsqrt(v + 1e-5)
    o_ref[...] = xb * s + (dm_ref[...] - m * s)


def _make_sc_gather(n_elems, per_w, nc, ns):
    mesh = plsc.VectorSubcoreMesh(core_axis_name="c", subcore_axis_name="s", num_cores=1)

    @functools.partial(
        pl.kernel,
        mesh=mesh,
        out_type=jax.ShapeDtypeStruct((n_elems,), jnp.float32),
        scratch_types=[
            pltpu.VMEM((per_w,), jnp.int32),
            pltpu.VMEM((per_w,), jnp.float32),
            pltpu.SemaphoreType.DMA,
        ],
    )
    def gather_k(wv_hbm, idx_hbm, out_hbm, idx_v, vals_v, sem):
        wid = lax.axis_index("s") * nc + lax.axis_index("c")
        base = wid * per_w
        pltpu.sync_copy(idx_hbm.at[pl.ds(base, per_w)], idx_v)
        pltpu.async_copy(wv_hbm.at[idx_v], vals_v, sem).wait()
        pltpu.sync_copy(vals_v, out_hbm.at[pl.ds(base, per_w)])

    return gather_k


def kernel(x, q, mean, std, Wk, Wv):
    b, c, h, w = x.shape
    buf, feat = Wk.shape
    bufd = Wv.shape[0]

    m_start = jnp.asarray(mean[1], jnp.int32) - c
    s_start = jnp.asarray(std[1], jnp.int32) - c
    m_start = m_start + (jnp.asarray(mean[0], jnp.int32) - m_start)
    s_start = s_start + (jnp.asarray(std[0], jnp.int32) - s_start)

    u = jax.random.uniform(jax.random.key(42), (b, buf), minval=1e-10, maxval=1.0)

    chunk = 2048
    nstream = 2
    nchunk = buf // (chunk * nstream)
    wk_specs = [
        pl.BlockSpec((chunk, feat), lambda k, s=s: (k * nstream + s, 0))
        for s in range(nstream)
    ]
    u_specs = [
        pl.BlockSpec((b, chunk), lambda k, s=s: (0, k * nstream + s))
        for s in range(nstream)
    ]
    idx_j = pl.pallas_call(
        functools.partial(_att_idx_body, buf=buf, chunk=chunk, nstream=nstream),
        grid=(nchunk,),
        in_specs=[pl.BlockSpec((b, feat), lambda k: (0, 0))]
        + wk_specs + u_specs,
        out_specs=pl.BlockSpec((b, bufd), lambda k: (0, 0)),
        out_shape=jax.ShapeDtypeStruct((b, bufd), jnp.int32),
        scratch_shapes=[
            pltpu.VMEM((b, 1), jnp.float32),
            pltpu.VMEM((b, 1), jnp.int32),
        ],
    )(q, *([Wk] * nstream), *([u] * nstream))

    n_elems = bufd * b
    info = plsc.get_sparse_core_info()
    nc, ns = 1, info.num_subcores
    per_w = n_elems // (nc * ns)
    vals = _make_sc_gather(n_elems, per_w, nc, ns)(
        Wv.reshape(-1), idx_j.reshape(-1)
    )
    ct = 48
    try:
        mo, so = int(m_start), int(s_start)
    except Exception:
        mo = so = None
    if mo is not None and mo % ct == 0 and so % ct == 0:
        dm_arr = ds_arr = vals.reshape(b, bufd, 1, 1)
        dm_spec = pl.BlockSpec((1, ct, 1, 1), lambda i, j: (i, mo // ct + j, 0, 0))
        ds_spec = pl.BlockSpec((1, ct, 1, 1), lambda i, j: (i, so // ct + j, 0, 0))
    else:
        di = vals.reshape(b, bufd)
        dm_arr = lax.dynamic_slice_in_dim(di, m_start, c, axis=1).reshape(b, c, 1, 1)
        ds_arr = lax.dynamic_slice_in_dim(di, s_start, c, axis=1).reshape(b, c, 1, 1)
        dm_spec = pl.BlockSpec((1, ct, 1, 1), lambda i, j: (i, j, 0, 0))
        ds_spec = pl.BlockSpec((1, ct, 1, 1), lambda i, j: (i, j, 0, 0))
    out = pl.pallas_call(
        _adain_body,
        grid=(b, c // ct),
        compiler_params=pltpu.CompilerParams(
            dimension_semantics=("parallel", "parallel"),
        ),
        in_specs=[
            pl.BlockSpec((1, ct, h, w), lambda i, j: (i, j, 0, 0)),
            dm_spec,
            ds_spec,
        ],
        out_specs=pl.BlockSpec((1, ct, h, w), lambda i, j: (i, j, 0, 0)),
        out_shape=jax.ShapeDtypeStruct((b, c, h, w), jnp.float32),
    )(x, dm_arr, ds_arr)
    return out
